# baseline (speedup 1.0000x reference)
import jax
import jax.numpy as jnp
from jax.experimental import pallas as pl

_NEG_POS_RATIO = 3


def _stage1(conf_ref, pred_ref, gt_ref, lab_ref, negv_ref, posce_ref, sl1_ref):
    c = conf_ref[0]
    lab = lab_ref[0, 0]
    m = jnp.max(c, axis=0)
    e = jnp.exp(c - m[None, :])
    lse = m + jnp.log(jnp.sum(e, axis=0))
    iota_c = jax.lax.broadcasted_iota(jnp.int32, c.shape, 0)
    c_at_label = jnp.sum(jnp.where(iota_c == lab[None, :], c, 0.0), axis=0)
    pos = lab > 0
    negv_ref[0, 0] = lse - c[0, :]
    posce_ref[0, 0] = jnp.where(pos, lse - c_at_label, 0.0)
    d = pred_ref[0] - gt_ref[0]
    ad = jnp.abs(d)
    sl1 = jnp.where(ad < 1.0, 0.5 * d * d, ad - 0.5)
    sl1_ref[0, 0] = jnp.where(pos, jnp.sum(sl1, axis=0), 0.0)


def _stage2(negv_ref, posce_ref, sl1_ref, lab_ref, out_ref):
    nv = negv_ref[:, 0, :]
    lab = lab_ref[:, 0, :]
    pos = lab > 0
    npos = jnp.sum(pos.astype(jnp.int32), axis=1, keepdims=True)
    nneg = nv.shape[1] - npos
    k = jnp.minimum(_NEG_POS_RATIO * npos, nneg)
    u = jnp.where(pos, 0, jax.lax.bitcast_convert_type(nv, jnp.int32))

    def body(i, prefix):
        test = prefix | jnp.left_shift(jnp.int32(1), 30 - i)
        cnt = jnp.sum((u >= test).astype(jnp.int32), axis=1, keepdims=True)
        return jnp.where(cnt >= k, test, prefix)

    t = jax.lax.fori_loop(0, 31, body, jnp.zeros_like(k))
    gt_mask = u > t
    cnt_gt = jnp.sum(gt_mask.astype(jnp.int32), axis=1, keepdims=True)
    sum_gt = jnp.sum(jnp.where(gt_mask, nv, 0.0), axis=1, keepdims=True)
    tf = jnp.where(k > 0, jax.lax.bitcast_convert_type(t, jnp.float32), 0.0)
    topk = jnp.where(k > 0, sum_gt + (k - cnt_gt).astype(jnp.float32) * tf, 0.0)

    class_total = jnp.sum(posce_ref[:, 0, :]) + jnp.sum(topk)
    sl1_total = jnp.sum(sl1_ref[:, 0, :])
    nptot = jnp.sum(npos).astype(jnp.float32)
    lb = sl1_total / nptot
    lc = class_total / nptot
    lanes = jax.lax.broadcasted_iota(jnp.int32, (1, 128), 1)
    out_ref[...] = jnp.where(
        lanes == 0, lb + lc, jnp.where(lanes == 1, lb, jnp.where(lanes == 2, lc, 0.0))
    )


def _run_stage1(conf_t, pred_t, gt_t, lab3):
    n, C, P = conf_t.shape
    return pl.pallas_call(
        _stage1,
        grid=(n,),
        in_specs=[
            pl.BlockSpec((1, C, P), lambda b: (b, 0, 0)),
            pl.BlockSpec((1, 4, P), lambda b: (b, 0, 0)),
            pl.BlockSpec((1, 4, P), lambda b: (b, 0, 0)),
            pl.BlockSpec((1, 1, P), lambda b: (b, 0, 0)),
        ],
        out_specs=[
            pl.BlockSpec((1, 1, P), lambda b: (b, 0, 0)),
            pl.BlockSpec((1, 1, P), lambda b: (b, 0, 0)),
            pl.BlockSpec((1, 1, P), lambda b: (b, 0, 0)),
        ],
        out_shape=[
            jax.ShapeDtypeStruct((n, 1, P), jnp.float32),
            jax.ShapeDtypeStruct((n, 1, P), jnp.float32),
            jax.ShapeDtypeStruct((n, 1, P), jnp.float32),
        ],
    )(conf_t, pred_t, gt_t, lab3)


_SC_ROWS = 18


def kernel(confidence, predicted_locations, labels, gt_locations):
    B, P, C = confidence.shape
    lab3 = labels.astype(jnp.int32).reshape(B, 1, P)
    S = _SC_ROWS
    zero = jax.lax.optimization_barrier(jnp.float32(0.0))
    conf_sc = jnp.transpose(confidence[:S], (0, 2, 1))
    conf_tc = jnp.transpose(confidence[S:], (0, 2, 1)) + zero
    pred_t = jnp.transpose(predicted_locations, (0, 2, 1)) + zero
    gt_t = jnp.transpose(gt_locations, (0, 2, 1)) + zero

    o_tc = _run_stage1(conf_tc, pred_t[S:], gt_t[S:], lab3[S:])
    o_sc = _run_stage1(conf_sc, pred_t[:S], gt_t[:S], lab3[:S])
    negv, posce, sl1 = (
        jnp.concatenate([a, b], axis=0) for a, b in zip(o_sc, o_tc)
    )

    out = pl.pallas_call(
        _stage2,
        out_shape=jax.ShapeDtypeStruct((1, 128), jnp.float32),
    )(negv, posce, sl1, lab3)

    return (out[0, 0], out[0, 1], out[0, 2])


# speedup vs baseline: 2.0002x; 2.0002x over previous
import functools

import jax
import jax.numpy as jnp
from jax import lax
from jax.experimental import pallas as pl
from jax.experimental.pallas import tpu as pltpu
from jax.experimental.pallas import tpu_sc as plsc

_NEG_POS_RATIO = 3
_P = 8732
_PP = 8736
_UNROLL = 6
_NCHUNK = _PP // (16 * _UNROLL)


def _stage1(conf_ref, pred_ref, gt_ref, lab_ref, negv_ref, negu_ref, posce_ref,
            sl1_ref, npos_ref):
    c = conf_ref[0]
    lab = lab_ref[0, 0]
    m = jnp.max(c, axis=0)
    e = jnp.exp(c - m[None, :])
    lse = m + jnp.log(jnp.sum(e, axis=0))
    iota_c = jax.lax.broadcasted_iota(jnp.int32, c.shape, 0)
    c_at_label = jnp.sum(jnp.where(iota_c == lab[None, :], c, 0.0), axis=0)
    pos = lab > 0
    nv = jnp.where(pos, 0.0, lse - c[0, :])
    negv_ref[0, 0, pl.ds(_PP - 8, 8)] = jnp.zeros((8,), jnp.float32)
    negv_ref[0, 0, pl.ds(0, _P)] = nv
    negu_ref[0, 0, pl.ds(_PP - 8, 8)] = jnp.zeros((8,), jnp.int32)
    negu_ref[0, 0, pl.ds(0, _P)] = jax.lax.bitcast_convert_type(nv, jnp.int32)
    posce_ref[0, 0] = jnp.where(pos, lse - c_at_label, 0.0)
    d = pred_ref[0] - gt_ref[0]
    ad = jnp.abs(d)
    sl1 = jnp.where(ad < 1.0, 0.5 * d * d, ad - 0.5)
    sl1_ref[0, 0] = jnp.where(pos, jnp.sum(sl1, axis=0), 0.0)
    npos_ref[0, 0] = jnp.full((128,), jnp.sum(pos.astype(jnp.float32)), jnp.float32)


def _sc_topk(negv_hbm, negu_hbm, npos_hbm, out_hbm, nv_v, nu_v, np_v, o_v):
    w = lax.axis_index("s") * 2 + lax.axis_index("c")
    pltpu.sync_copy(negv_hbm.at[w], nv_v)
    pltpu.sync_copy(negu_hbm.at[w], nu_v)
    pltpu.sync_copy(npos_hbm.at[w], np_v)
    npos = np_v[...].astype(jnp.int32)
    k = jnp.minimum(_NEG_POS_RATIO * npos, _P - npos)
    one = jnp.full((16,), 1, jnp.int32)
    zero = jnp.zeros((16,), jnp.int32)

    gd = jax.lax.GatherDimensionNumbers(
        offset_dims=(), collapsed_slice_dims=(0,), start_index_map=(0,)
    )

    def lane_sum(x):
        for sh in (8, 4, 2, 1):
            idx = jax.lax.iota(jnp.int32, 16) ^ sh
            x = x + jax.lax.gather(
                x, idx[:, None], gd, (1,),
                mode=jax.lax.GatherScatterMode.PROMISE_IN_BOUNDS,
            )
        return x

    def count_ge(test):
        def body(i, acc):
            base = i * (16 * _UNROLL)
            for j in range(_UNROLL):
                u = nu_v[pl.ds(base + j * 16, 16)]
                acc = acc + jnp.where(u >= test, one, zero)
            return acc
        return lane_sum(lax.fori_loop(0, _NCHUNK, body, zero))

    def descend(b, prefix):
        test = prefix | lax.shift_left(one, 30 - b)
        return jnp.where(count_ge(test) >= k, test, prefix)

    t = lax.fori_loop(0, 31, descend, jnp.zeros((16,), jnp.int32))

    def tail(i, carry):
        sacc, cacc, macc = carry
        base = i * (16 * _UNROLL)
        for j in range(_UNROLL):
            v = nv_v[pl.ds(base + j * 16, 16)]
            u = nu_v[pl.ds(base + j * 16, 16)]
            gt = u > t
            sacc = sacc + jnp.where(gt, v, 0.0)
            cacc = cacc + jnp.where(gt, one, zero)
            macc = jnp.maximum(macc, jnp.where(gt, -1.0, v))
        return sacc, cacc, macc

    sacc, cacc, macc = lax.fori_loop(
        0, _NCHUNK, tail,
        (jnp.zeros((16,), jnp.float32), jnp.zeros((16,), jnp.int32),
         jnp.full((16,), -1.0, jnp.float32)),
    )
    o_v[pl.ds(0, 16)] = sacc
    o_v[pl.ds(16, 16)] = macc
    o_v[pl.ds(32, 16)] = (k - lane_sum(cacc)).astype(jnp.float32)
    pltpu.sync_copy(o_v, out_hbm.at[w])


def _final(posce_ref, sl1_ref, lab_ref, topk_ref, out_ref):
    pos_i = (lab_ref[:, 0, :] > 0).astype(jnp.float32)
    nptot = jnp.sum(pos_i)
    tk = topk_ref[...]
    tval = jnp.max(tk[:, 16:32], axis=1, keepdims=True)
    topk_total = jnp.sum(tk[:, 0:16]) + jnp.sum(tval * tk[:, 32:33])
    class_total = jnp.sum(posce_ref[:, 0, :]) + topk_total
    sl1_total = jnp.sum(sl1_ref[:, 0, :])
    lb = sl1_total / nptot
    lc = class_total / nptot
    lanes = jax.lax.broadcasted_iota(jnp.int32, (1, 128), 1)
    out_ref[...] = jnp.where(
        lanes == 0, lb + lc, jnp.where(lanes == 1, lb, jnp.where(lanes == 2, lc, 0.0))
    )


def kernel(confidence, predicted_locations, labels, gt_locations):
    B, P, C = confidence.shape
    lab3 = labels.astype(jnp.int32).reshape(B, 1, P)
    zero = jax.lax.optimization_barrier(jnp.float32(0.0))
    conf_t = jnp.transpose(confidence, (0, 2, 1)) + zero
    pred_t = jnp.transpose(predicted_locations, (0, 2, 1)) + zero
    gt_t = jnp.transpose(gt_locations, (0, 2, 1)) + zero

    negv, negu, posce, sl1, nposv = pl.pallas_call(
        _stage1,
        grid=(B,),
        in_specs=[
            pl.BlockSpec((1, C, P), lambda b: (b, 0, 0)),
            pl.BlockSpec((1, 4, P), lambda b: (b, 0, 0)),
            pl.BlockSpec((1, 4, P), lambda b: (b, 0, 0)),
            pl.BlockSpec((1, 1, P), lambda b: (b, 0, 0)),
        ],
        out_specs=[
            pl.BlockSpec((1, 1, _PP), lambda b: (b, 0, 0)),
            pl.BlockSpec((1, 1, _PP), lambda b: (b, 0, 0)),
            pl.BlockSpec((1, 1, P), lambda b: (b, 0, 0)),
            pl.BlockSpec((1, 1, P), lambda b: (b, 0, 0)),
            pl.BlockSpec((1, 1, 128), lambda b: (b, 0, 0)),
        ],
        out_shape=[
            jax.ShapeDtypeStruct((B, 1, _PP), jnp.float32),
            jax.ShapeDtypeStruct((B, 1, _PP), jnp.int32),
            jax.ShapeDtypeStruct((B, 1, P), jnp.float32),
            jax.ShapeDtypeStruct((B, 1, P), jnp.float32),
            jax.ShapeDtypeStruct((B, 1, 128), jnp.float32),
        ],
    )(conf_t, pred_t, gt_t, lab3)

    sc_kernel = functools.partial(
        pl.kernel,
        out_type=jax.ShapeDtypeStruct((B, 48), jnp.float32),
        mesh=plsc.VectorSubcoreMesh(core_axis_name="c", subcore_axis_name="s"),
        scratch_types=[
            pltpu.VMEM((_PP,), jnp.float32),
            pltpu.VMEM((_PP,), jnp.int32),
            pltpu.VMEM((16,), jnp.float32),
            pltpu.VMEM((48,), jnp.float32),
        ],
    )(_sc_topk)
    topk = sc_kernel(
        negv.reshape(B, _PP), negu.reshape(B, _PP), nposv.reshape(B, 128)[:, :16]
    )

    out = pl.pallas_call(
        _final,
        out_shape=jax.ShapeDtypeStruct((1, 128), jnp.float32),
    )(posce, sl1, lab3, topk)

    return (out[0, 0], out[0, 1], out[0, 2])
